# Initial kernel scaffold; baseline (speedup 1.0000x reference)
import jax
import jax.numpy as jnp
from jax.experimental import pallas as pl


def kernel(x, edge_index, cand_idx, W, b, Wc, bc):
    raise NotImplementedError("write your pallas kernel here")


# speedup vs baseline: 36.6988x; 36.6988x over previous
import functools

import jax
import jax.numpy as jnp
from jax import lax
from jax.experimental import pallas as pl
from jax.experimental.pallas import tpu as pltpu
from jax.experimental.pallas import tpu_sc as plsc

N = 10000
E = 320000
D = 128
C = 64

NCORE = 2
NSUB = 16
NW = NCORE * NSUB
EPT = E // NW
K = 16
CAP = EPT + 32
NACC = 72
TRASH = 64


def _sc_body(x_hbm, src_hbm, dst_hbm, cand_hbm,
             acc_out, deg_out, slot_out,
             dstv, srcv, mapv, candv, msrc, mslot,
             rows, degt, slotv, acc_sh, sem):
    cid = lax.axis_index("c")
    sid = lax.axis_index("s")
    wid = cid * NSUB + sid
    base = wid * EPT

    pltpu.sync_copy(dst_hbm.at[pl.ds(base, EPT)], dstv)
    pltpu.sync_copy(src_hbm.at[pl.ds(base, EPT)], srcv)
    pltpu.sync_copy(cand_hbm, candv)

    zf = jnp.zeros((16,), jnp.float32)
    of = jnp.full((16,), 1.0, jnp.float32)
    zi = jnp.zeros((16,), jnp.int32)
    ti = jnp.full((16,), TRASH, jnp.int32)
    lanes = lax.iota(jnp.int32, 16)

    def zrow_body(r, c):
        degt[r, :] = zf
        for g in range(D // 16):
            rows[r, pl.ds(g * 16, 16)] = zf
        return c

    lax.fori_loop(0, NACC, zrow_body, 0)

    @pl.when(sid == 0)
    def _():
        pltpu.sync_copy(rows, acc_sh)

    def map_zero(i, c):
        mapv[pl.ds(i * 16, 16)] = zi
        return c

    lax.fori_loop(0, N // 16, map_zero, 0)
    for g in range(C // 16):
        vals = candv[pl.ds(g * 16, 16)]
        slots = lanes + (g * 16 + 1)
        for l in range(16):
            plsc.store_scatter(mapv, [vals], slots, mask=lanes == l)

    def match_body(i, off):
        d = dstv[pl.ds(i * 16, 16)]
        s = srcv[pl.ds(i * 16, 16)]
        slot = plsc.load_gather(mapv, [d])
        m = slot > 0
        plsc.store_compressed(msrc.at[pl.ds(off, 16)], s, mask=m)
        plsc.store_compressed(mslot.at[pl.ds(off, 16)], slot - 1, mask=m)
        plsc.addupdate_scatter(degt, [slot - 1, lanes], of, mask=m)
        return off + jnp.sum(m.astype(jnp.int32))

    off = lax.fori_loop(0, EPT // 16, match_body, jnp.int32(0))

    for t in range(K // 16):
        msrc[pl.ds(off + t * 16, 16)] = zi
        mslot[pl.ds(off + t * 16, 16)] = ti

    plsc.subcore_barrier()

    nch = (off + (K - 1)) // K

    def chunk_body(ch, c):
        sv = msrc[pl.ds(ch * K, K)]
        tv = mslot[pl.ds(ch * K, K)]
        pltpu.async_copy(x_hbm.at[sv], rows.at[pl.ds(0, K)], sem).wait()
        pltpu.sync_copy(rows.at[pl.ds(0, K)], acc_sh.at[tv], add=True)
        return c

    lax.fori_loop(0, nch, chunk_body, 0)

    pltpu.sync_copy(degt, deg_out.at[wid])

    plsc.subcore_barrier()

    @pl.when(sid == 0)
    def _():
        for g in range(C // 16):
            vals = candv[pl.ds(g * 16, 16)]
            slotv[pl.ds(g * 16, 16)] = plsc.load_gather(mapv, [vals])
        pltpu.sync_copy(slotv, slot_out.at[cid])
        pltpu.sync_copy(acc_sh, rows)
        pltpu.sync_copy(rows, acc_out.at[cid])


def _tc_body(acc_ref, deg_ref, slot_ref, w_ref, b_ref, wc_ref, bc_ref, o_ref):
    acc = acc_ref[0] + acc_ref[1]
    deg = jnp.sum(deg_ref[...], axis=0)
    sl0 = slot_ref[0:1, :] - 1
    pt = (lax.broadcasted_iota(jnp.int32, (NACC, C), 0)
          == jnp.broadcast_to(sl0, (NACC, C))).astype(jnp.float32)
    dims = (((0,), (0,)), ((), ()))
    agg = lax.dot_general(pt, acc, dims,
                          preferred_element_type=jnp.float32)
    d64 = jnp.sum(lax.dot_general(pt, deg, dims,
                                  preferred_element_type=jnp.float32),
                  axis=1, keepdims=True)
    mean = agg / jnp.maximum(d64, 1.0)
    h = jnp.maximum(
        jnp.dot(mean, w_ref[...], preferred_element_type=jnp.float32)
        + b_ref[...], 0.0)
    out = lax.dot_general(wc_ref[...], h, (((0,), (1,)), ((), ())),
                          preferred_element_type=jnp.float32)
    o_ref[...] = out + bc_ref[...]


def kernel(x, edge_index, cand_idx, W, b, Wc, bc):
    src = edge_index[0]
    dst = edge_index[1]
    mesh = plsc.VectorSubcoreMesh(core_axis_name="c", subcore_axis_name="s")
    sc = pl.kernel(
        _sc_body,
        out_type=[
            jax.ShapeDtypeStruct((NCORE, NACC, D), jnp.float32),
            jax.ShapeDtypeStruct((NW, NACC, 16), jnp.float32),
            jax.ShapeDtypeStruct((NCORE, C), jnp.int32),
        ],
        mesh=mesh,
        compiler_params=pltpu.CompilerParams(needs_layout_passes=False),
        scratch_types=[
            pltpu.VMEM((EPT,), jnp.int32),
            pltpu.VMEM((EPT,), jnp.int32),
            pltpu.VMEM((N,), jnp.int32),
            pltpu.VMEM((C,), jnp.int32),
            pltpu.VMEM((CAP,), jnp.int32),
            pltpu.VMEM((CAP,), jnp.int32),
            pltpu.VMEM((NACC, D), jnp.float32),
            pltpu.VMEM((NACC, 16), jnp.float32),
            pltpu.VMEM((C,), jnp.int32),
            pltpu.VMEM_SHARED((NACC, D), jnp.float32),
            pltpu.SemaphoreType.DMA,
        ],
    )
    acc2, deg2, slot2 = sc(x, src, dst, cand_idx)
    out = pl.pallas_call(
        _tc_body,
        out_shape=jax.ShapeDtypeStruct((1, C), jnp.float32),
    )(acc2, deg2, slot2, W, b.reshape(1, D), Wc, bc.reshape(1, 1))
    return out.reshape(C)


# speedup vs baseline: 36.7282x; 1.0008x over previous
import functools

import jax
import jax.numpy as jnp
from jax import lax
from jax.experimental import pallas as pl
from jax.experimental.pallas import tpu as pltpu
from jax.experimental.pallas import tpu_sc as plsc

N = 10000
E = 320000
D = 128
C = 64

NCORE = 2
NSUB = 16
NW = NCORE * NSUB
EPT = E // NW
K = 16
CAP = EPT + 32
NACC = 72
TRASH = 64


def _sc_body(x_hbm, src_hbm, dst_hbm, cand_hbm,
             acc_out, deg_out, slot_out,
             dstv, srcv, mapv, candv, msrc, mslot,
             rows, degt, slotv, acc_sh, sem):
    cid = lax.axis_index("c")
    sid = lax.axis_index("s")
    wid = cid * NSUB + sid
    base = wid * EPT

    cdst = pltpu.async_copy(dst_hbm.at[pl.ds(base, EPT)], dstv, sem)
    csrc = pltpu.async_copy(src_hbm.at[pl.ds(base, EPT)], srcv, sem)
    pltpu.sync_copy(cand_hbm, candv)

    zf = jnp.zeros((16,), jnp.float32)
    of = jnp.full((16,), 1.0, jnp.float32)
    zi = jnp.zeros((16,), jnp.int32)
    ti = jnp.full((16,), TRASH, jnp.int32)
    lanes = lax.iota(jnp.int32, 16)

    def zrow_body(r, c):
        degt[r, :] = zf
        for g in range(D // 16):
            rows[r, pl.ds(g * 16, 16)] = zf
        return c

    lax.fori_loop(0, NACC, zrow_body, 0)

    @pl.when(sid == 0)
    def _():
        pltpu.sync_copy(rows, acc_sh)

    def map_zero(i, c):
        mapv[pl.ds(i * 16, 16)] = zi
        return c

    lax.fori_loop(0, N // 16, map_zero, 0)
    for g in range(C // 16):
        vals = candv[pl.ds(g * 16, 16)]
        slots = lanes + (g * 16 + 1)
        for l in range(16):
            plsc.store_scatter(mapv, [vals], slots, mask=lanes == l)

    cdst.wait()
    csrc.wait()

    def match_body(i, off):
        d = dstv[pl.ds(i * 16, 16)]
        s = srcv[pl.ds(i * 16, 16)]
        slot = plsc.load_gather(mapv, [d])
        m = slot > 0
        plsc.store_compressed(msrc.at[pl.ds(off, 16)], s, mask=m)
        plsc.store_compressed(mslot.at[pl.ds(off, 16)], slot - 1, mask=m)
        plsc.addupdate_scatter(degt, [slot - 1, lanes], of, mask=m)
        return off + plsc.all_reduce_population_count(m)[0]

    off = lax.fori_loop(0, EPT // 16, match_body, jnp.int32(0))

    for t in range(K // 16):
        msrc[pl.ds(off + t * 16, 16)] = zi
        mslot[pl.ds(off + t * 16, 16)] = ti

    plsc.subcore_barrier()

    nch = (off + (K - 1)) // K

    def chunk_body(ch, c):
        sv = msrc[pl.ds(ch * K, K)]
        tv = mslot[pl.ds(ch * K, K)]
        pltpu.async_copy(x_hbm.at[sv], rows.at[pl.ds(0, K)], sem).wait()
        pltpu.sync_copy(rows.at[pl.ds(0, K)], acc_sh.at[tv], add=True)
        return c

    lax.fori_loop(0, nch, chunk_body, 0)

    pltpu.sync_copy(degt, deg_out.at[wid])

    plsc.subcore_barrier()

    @pl.when(sid == 0)
    def _():
        for g in range(C // 16):
            vals = candv[pl.ds(g * 16, 16)]
            slotv[pl.ds(g * 16, 16)] = plsc.load_gather(mapv, [vals])
        pltpu.sync_copy(slotv, slot_out.at[cid])
        pltpu.sync_copy(acc_sh, rows)
        pltpu.sync_copy(rows, acc_out.at[cid])


def _tc_body(acc_ref, deg_ref, slot_ref, w_ref, b_ref, wc_ref, bc_ref, o_ref):
    acc = acc_ref[0] + acc_ref[1]
    deg = jnp.sum(deg_ref[...], axis=0)
    sl0 = slot_ref[0:1, :] - 1
    pt = (lax.broadcasted_iota(jnp.int32, (NACC, C), 0)
          == jnp.broadcast_to(sl0, (NACC, C))).astype(jnp.float32)
    dims = (((0,), (0,)), ((), ()))
    agg = lax.dot_general(pt, acc, dims,
                          preferred_element_type=jnp.float32)
    d64 = jnp.sum(lax.dot_general(pt, deg, dims,
                                  preferred_element_type=jnp.float32),
                  axis=1, keepdims=True)
    mean = agg / jnp.maximum(d64, 1.0)
    h = jnp.maximum(
        jnp.dot(mean, w_ref[...], preferred_element_type=jnp.float32)
        + b_ref[...], 0.0)
    out = lax.dot_general(wc_ref[...], h, (((0,), (1,)), ((), ())),
                          preferred_element_type=jnp.float32)
    o_ref[...] = out + bc_ref[...]


def kernel(x, edge_index, cand_idx, W, b, Wc, bc):
    src = edge_index[0]
    dst = edge_index[1]
    mesh = plsc.VectorSubcoreMesh(core_axis_name="c", subcore_axis_name="s")
    sc = pl.kernel(
        _sc_body,
        out_type=[
            jax.ShapeDtypeStruct((NCORE, NACC, D), jnp.float32),
            jax.ShapeDtypeStruct((NW, NACC, 16), jnp.float32),
            jax.ShapeDtypeStruct((NCORE, C), jnp.int32),
        ],
        mesh=mesh,
        compiler_params=pltpu.CompilerParams(needs_layout_passes=False),
        scratch_types=[
            pltpu.VMEM((EPT,), jnp.int32),
            pltpu.VMEM((EPT,), jnp.int32),
            pltpu.VMEM((N,), jnp.int32),
            pltpu.VMEM((C,), jnp.int32),
            pltpu.VMEM((CAP,), jnp.int32),
            pltpu.VMEM((CAP,), jnp.int32),
            pltpu.VMEM((NACC, D), jnp.float32),
            pltpu.VMEM((NACC, 16), jnp.float32),
            pltpu.VMEM((C,), jnp.int32),
            pltpu.VMEM_SHARED((NACC, D), jnp.float32),
            pltpu.SemaphoreType.DMA,
        ],
    )
    acc2, deg2, slot2 = sc(x, src, dst, cand_idx)
    out = pl.pallas_call(
        _tc_body,
        out_shape=jax.ShapeDtypeStruct((1, C), jnp.float32),
    )(acc2, deg2, slot2, W, b.reshape(1, D), Wc, bc.reshape(1, 1))
    return out.reshape(C)


# speedup vs baseline: 36.7585x; 1.0008x over previous
import functools

import jax
import jax.numpy as jnp
from jax import lax
from jax.experimental import pallas as pl
from jax.experimental.pallas import tpu as pltpu
from jax.experimental.pallas import tpu_sc as plsc

N = 10000
E = 320000
D = 128
C = 64

NCORE = 2
NSUB = 16
NW = NCORE * NSUB
EPT = E // NW
K = 16
CAP = EPT + 32
NACC = 72
TRASH = 64


def _sc_body(x_hbm, src_hbm, dst_hbm, cand_hbm,
             acc_out, deg_out, slot_out,
             dstv, srcv, mapv, candv, msrc, mslot,
             rows, degt, slotv, acc_sh, sem):
    cid = lax.axis_index("c")
    sid = lax.axis_index("s")
    wid = cid * NSUB + sid
    base = wid * EPT

    cdst = pltpu.async_copy(dst_hbm.at[pl.ds(base, EPT)], dstv, sem)
    csrc = pltpu.async_copy(src_hbm.at[pl.ds(base, EPT)], srcv, sem)
    pltpu.sync_copy(cand_hbm, candv)

    zf = jnp.zeros((16,), jnp.float32)
    of = jnp.full((16,), 1.0, jnp.float32)
    zi = jnp.zeros((16,), jnp.int32)
    ti = jnp.full((16,), TRASH, jnp.int32)
    lanes = lax.iota(jnp.int32, 16)

    def zrow_body(r, c):
        degt[r, :] = zf
        for g in range(D // 16):
            rows[r, pl.ds(g * 16, 16)] = zf
        return c

    lax.fori_loop(0, NACC, zrow_body, 0)

    @pl.when(sid == 0)
    def _():
        pltpu.sync_copy(rows, acc_sh)

    def map_zero(i, c):
        mapv[pl.ds(i * 16, 16)] = zi
        return c

    with jax.named_scope("ph_mapzero"):
        lax.fori_loop(0, N // 16, map_zero, 0)
    for g in range(C // 16):
        vals = candv[pl.ds(g * 16, 16)]
        slots = lanes + (g * 16 + 1)
        for l in range(16):
            plsc.store_scatter(mapv, [vals], slots, mask=lanes == l)

    with jax.named_scope("ph_dmawait"):
        cdst.wait()
        csrc.wait()

    def match_body(i, off):
        d = dstv[pl.ds(i * 16, 16)]
        s = srcv[pl.ds(i * 16, 16)]
        slot = plsc.load_gather(mapv, [d])
        m = slot > 0
        plsc.store_compressed(msrc.at[pl.ds(off, 16)], s, mask=m)
        plsc.store_compressed(mslot.at[pl.ds(off, 16)], slot - 1, mask=m)
        plsc.addupdate_scatter(degt, [slot - 1, lanes], of, mask=m)
        return off + plsc.all_reduce_population_count(m)[0]

    with jax.named_scope("ph_match"):
        off = lax.fori_loop(0, EPT // 16, match_body, jnp.int32(0))

    for t in range(K // 16):
        msrc[pl.ds(off + t * 16, 16)] = zi
        mslot[pl.ds(off + t * 16, 16)] = ti

    plsc.subcore_barrier()

    nch = (off + (K - 1)) // K

    def chunk_body(ch, c):
        sv = msrc[pl.ds(ch * K, K)]
        tv = mslot[pl.ds(ch * K, K)]
        pltpu.async_copy(x_hbm.at[sv], rows.at[pl.ds(0, K)], sem).wait()
        pltpu.sync_copy(rows.at[pl.ds(0, K)], acc_sh.at[tv], add=True)
        return c

    with jax.named_scope("ph_chunks"):
        lax.fori_loop(0, nch, chunk_body, 0)

    pltpu.sync_copy(degt, deg_out.at[wid])

    plsc.subcore_barrier()

    @pl.when(sid == 0)
    def _():
        for g in range(C // 16):
            vals = candv[pl.ds(g * 16, 16)]
            slotv[pl.ds(g * 16, 16)] = plsc.load_gather(mapv, [vals])
        pltpu.sync_copy(slotv, slot_out.at[cid])
        pltpu.sync_copy(acc_sh, rows)
        pltpu.sync_copy(rows, acc_out.at[cid])


def _tc_body(acc_ref, deg_ref, slot_ref, w_ref, b_ref, wc_ref, bc_ref, o_ref):
    acc = acc_ref[0] + acc_ref[1]
    deg = jnp.sum(deg_ref[...], axis=0)
    sl0 = slot_ref[0:1, :] - 1
    pt = (lax.broadcasted_iota(jnp.int32, (NACC, C), 0)
          == jnp.broadcast_to(sl0, (NACC, C))).astype(jnp.float32)
    dims = (((0,), (0,)), ((), ()))
    agg = lax.dot_general(pt, acc, dims,
                          preferred_element_type=jnp.float32)
    d64 = jnp.sum(lax.dot_general(pt, deg, dims,
                                  preferred_element_type=jnp.float32),
                  axis=1, keepdims=True)
    mean = agg / jnp.maximum(d64, 1.0)
    h = jnp.maximum(
        jnp.dot(mean, w_ref[...], preferred_element_type=jnp.float32)
        + b_ref[...], 0.0)
    out = lax.dot_general(wc_ref[...], h, (((0,), (1,)), ((), ())),
                          preferred_element_type=jnp.float32)
    o_ref[...] = out + bc_ref[...]


def kernel(x, edge_index, cand_idx, W, b, Wc, bc):
    src = edge_index[0]
    dst = edge_index[1]
    mesh = plsc.VectorSubcoreMesh(core_axis_name="c", subcore_axis_name="s")
    sc = pl.kernel(
        _sc_body,
        out_type=[
            jax.ShapeDtypeStruct((NCORE, NACC, D), jnp.float32),
            jax.ShapeDtypeStruct((NW, NACC, 16), jnp.float32),
            jax.ShapeDtypeStruct((NCORE, C), jnp.int32),
        ],
        mesh=mesh,
        compiler_params=pltpu.CompilerParams(needs_layout_passes=False),
        scratch_types=[
            pltpu.VMEM((EPT,), jnp.int32),
            pltpu.VMEM((EPT,), jnp.int32),
            pltpu.VMEM((N,), jnp.int32),
            pltpu.VMEM((C,), jnp.int32),
            pltpu.VMEM((CAP,), jnp.int32),
            pltpu.VMEM((CAP,), jnp.int32),
            pltpu.VMEM((NACC, D), jnp.float32),
            pltpu.VMEM((NACC, 16), jnp.float32),
            pltpu.VMEM((C,), jnp.int32),
            pltpu.VMEM_SHARED((NACC, D), jnp.float32),
            pltpu.SemaphoreType.DMA,
        ],
    )
    acc2, deg2, slot2 = sc(x, src, dst, cand_idx)
    out = pl.pallas_call(
        _tc_body,
        out_shape=jax.ShapeDtypeStruct((1, C), jnp.float32),
    )(acc2, deg2, slot2, W, b.reshape(1, D), Wc, bc.reshape(1, 1))
    return out.reshape(C)


# speedup vs baseline: 49.6746x; 1.3514x over previous
import functools

import jax
import jax.numpy as jnp
from jax import lax
from jax.experimental import pallas as pl
from jax.experimental.pallas import tpu as pltpu
from jax.experimental.pallas import tpu_sc as plsc

N = 10000
E = 320000
D = 128
C = 64

NCORE = 2
NSUB = 16
NW = NCORE * NSUB
EPT = E // NW
K = 16
CAP = EPT + 32
NACC = 72
TRASH = 64


def _sc_body(x_hbm, edge_hbm, cand_hbm,
             acc_out, deg_out, slot_out,
             dstv, srcv, mapv, candv, mpair,
             rows, degt, slotv, acc_sh, sem):
    cid = lax.axis_index("c")
    sid = lax.axis_index("s")
    wid = cid * NSUB + sid
    base = wid * EPT

    cdst = pltpu.async_copy(edge_hbm.at[pl.ds(E + base, EPT)], dstv, sem)
    csrc = pltpu.async_copy(edge_hbm.at[pl.ds(base, EPT)], srcv, sem)
    pltpu.sync_copy(cand_hbm, candv)

    zf = jnp.zeros((16,), jnp.float32)
    of = jnp.full((16,), 1.0, jnp.float32)
    zi = jnp.zeros((16,), jnp.int32)
    ti = jnp.full((16,), TRASH, jnp.int32)
    lanes = lax.iota(jnp.int32, 16)

    def zrow_body(r, c):
        degt[r, :] = zf
        for g in range(D // 16):
            rows[r, pl.ds(g * 16, 16)] = zf
        return c

    lax.fori_loop(0, NACC, zrow_body, 0)

    @pl.when(sid == 0)
    def _():
        pltpu.sync_copy(rows, acc_sh)

    def map_zero(i, c):
        for u in range(5):
            mapv[pl.ds(i * 80 + u * 16, 16)] = zi
        return c

    with jax.named_scope("ph_mapzero"):
        lax.fori_loop(0, N // 80, map_zero, 0)
    for g in range(C // 16):
        vals = candv[pl.ds(g * 16, 16)]
        slots = lanes + (g * 16 + 1)
        for l in range(16):
            plsc.store_scatter(mapv, [vals], slots, mask=lanes == l)

    with jax.named_scope("ph_dmawait"):
        cdst.wait()
        csrc.wait()

    def match_body(i, off):
        d = dstv[pl.ds(i * 16, 16)]
        s = srcv[pl.ds(i * 16, 16)]
        slot = plsc.load_gather(mapv, [d])
        m = slot > 0
        packed = (s << 7) | (slot - 1)
        plsc.store_compressed(mpair.at[pl.ds(off, 16)], packed, mask=m)
        return off + plsc.all_reduce_population_count(m)[0]

    with jax.named_scope("ph_match"):
        off = lax.fori_loop(0, EPT // 16, match_body, jnp.int32(0))

    for t in range(K // 16):
        mpair[pl.ds(off + t * 16, 16)] = ti

    plsc.subcore_barrier()

    nch = (off + (K - 1)) // K

    def chunk_body(ch, c):
        packed = mpair[pl.ds(ch * K, K)]
        sv = packed >> 7
        tv = packed & 127
        pltpu.async_copy(x_hbm.at[sv], rows.at[pl.ds(0, K)], sem).wait()
        pltpu.sync_copy(rows.at[pl.ds(0, K)], acc_sh.at[tv], add=True)
        plsc.addupdate_scatter(degt, [tv, lanes], of, mask=None)
        return c

    with jax.named_scope("ph_chunks"):
        lax.fori_loop(0, nch, chunk_body, 0)

    pltpu.sync_copy(degt, deg_out.at[wid])

    plsc.subcore_barrier()

    @pl.when(sid == 0)
    def _():
        for g in range(C // 16):
            vals = candv[pl.ds(g * 16, 16)]
            slotv[pl.ds(g * 16, 16)] = plsc.load_gather(mapv, [vals])
        pltpu.sync_copy(slotv, slot_out.at[cid])
        pltpu.sync_copy(acc_sh, rows)
        pltpu.sync_copy(rows, acc_out.at[cid])


def _tc_body(acc_ref, deg_ref, slot_ref, w_ref, b_ref, wc_ref, bc_ref, o_ref):
    acc = acc_ref[0] + acc_ref[1]
    deg = jnp.sum(deg_ref[...], axis=0)
    sl0 = slot_ref[0:1, :] - 1
    pt = (lax.broadcasted_iota(jnp.int32, (NACC, C), 0)
          == jnp.broadcast_to(sl0, (NACC, C))).astype(jnp.float32)
    dims = (((0,), (0,)), ((), ()))
    agg = lax.dot_general(pt, acc, dims,
                          preferred_element_type=jnp.float32)
    d64 = jnp.sum(lax.dot_general(pt, deg, dims,
                                  preferred_element_type=jnp.float32),
                  axis=1, keepdims=True)
    mean = agg / jnp.maximum(d64, 1.0)
    h = jnp.maximum(
        jnp.dot(mean, w_ref[...], preferred_element_type=jnp.float32)
        + b_ref[...], 0.0)
    out = lax.dot_general(wc_ref[...], h, (((0,), (1,)), ((), ())),
                          preferred_element_type=jnp.float32)
    o_ref[...] = out + bc_ref[...]


def kernel(x, edge_index, cand_idx, W, b, Wc, bc):
    mesh = plsc.VectorSubcoreMesh(core_axis_name="c", subcore_axis_name="s")
    sc = pl.kernel(
        _sc_body,
        out_type=[
            jax.ShapeDtypeStruct((NCORE, NACC, D), jnp.float32),
            jax.ShapeDtypeStruct((NW, NACC, 16), jnp.float32),
            jax.ShapeDtypeStruct((NCORE, C), jnp.int32),
        ],
        mesh=mesh,
        compiler_params=pltpu.CompilerParams(needs_layout_passes=False),
        scratch_types=[
            pltpu.VMEM((EPT,), jnp.int32),
            pltpu.VMEM((EPT,), jnp.int32),
            pltpu.VMEM((N,), jnp.int32),
            pltpu.VMEM((C,), jnp.int32),
            pltpu.VMEM((CAP,), jnp.int32),
            pltpu.VMEM((NACC, D), jnp.float32),
            pltpu.VMEM((NACC, 16), jnp.float32),
            pltpu.VMEM((C,), jnp.int32),
            pltpu.VMEM_SHARED((NACC, D), jnp.float32),
            pltpu.SemaphoreType.DMA,
        ],
    )
    acc2, deg2, slot2 = sc(x, edge_index.reshape(2 * E), cand_idx)
    out = pl.pallas_call(
        _tc_body,
        out_shape=jax.ShapeDtypeStruct((1, C), jnp.float32),
    )(acc2, deg2, slot2, W, b.reshape(1, D), Wc, bc.reshape(1, 1))
    return out.reshape(C)
